# baseline (speedup 1.0000x reference)
import functools

import jax
import jax.numpy as jnp
from jax import lax
from jax.experimental import pallas as pl
from jax.experimental.pallas import tpu as pltpu
from jax.experimental.pallas import tpu_sc as plsc

NUM_EMB = 1000000
DIM = 32
ND = 13
NSF = 26
B = 4096
NROW = NSF + ND

NC = 2
NS = 16
NW = NC * NS
BPW = B // NW
G = 128
NG_SP = BPW * NSF // G
NG_DE = BPW * ND // G

TCH = 32000


def _dense_tc(x, w, bias):
    def body(x_ref, w_ref, b_ref, o_ref):
        o_ref[...] = (
            lax.dot_general(
                x_ref[...], w_ref[...],
                dimension_numbers=(((1,), (1,)), ((), ())),
                preferred_element_type=jnp.float32,
            )
            + b_ref[...]
        )

    return pl.pallas_call(
        body,
        out_shape=jax.ShapeDtypeStruct((B, ND * DIM), jnp.float32),
    )(x, w, bias.reshape(1, ND * DIM))


def _transpose_tc(tt2):
    NFULL = NUM_EMB // TCH
    TAIL = NUM_EMB - TCH * NFULL

    def body(x_hbm, o_hbm, xv, ov, xt, ot, isems, osems, tsem):
        def fire_in(c):
            pltpu.make_async_copy(
                x_hbm.at[:, pl.ds(c * TCH, TCH)],
                xv.at[c % 2],
                isems.at[c % 2],
            ).start()

        fire_in(0)
        fire_in(1)
        pltpu.make_async_copy(
            x_hbm.at[:, pl.ds(NFULL * TCH, TAIL)], xt, tsem
        ).start()
        for c in range(NFULL):
            kb = c % 2
            pltpu.make_async_copy(
                x_hbm.at[:, pl.ds(c * TCH, TCH)], xv.at[kb], isems.at[kb]
            ).wait()
            q = TCH // 4
            y = jnp.concatenate(
                [xv[kb][:, j * q:(j + 1) * q] for j in range(4)], axis=0
            ).T
            if c >= 2:
                pltpu.make_async_copy(
                    ov.at[kb],
                    o_hbm.at[pl.ds((c - 2) * (TCH // 4), TCH // 4)],
                    osems.at[kb],
                ).wait()
            ov[kb] = y
            pltpu.make_async_copy(
                ov.at[kb],
                o_hbm.at[pl.ds(c * (TCH // 4), TCH // 4)],
                osems.at[kb],
            ).start()
            if c + 2 < NFULL:
                fire_in(c + 2)
        pltpu.make_async_copy(
            x_hbm.at[:, pl.ds(NFULL * TCH, TAIL)], xt, tsem
        ).wait()
        qt = TAIL // 4
        ot[...] = jnp.concatenate(
            [xt[:, j * qt:(j + 1) * qt] for j in range(4)], axis=0
        ).T
        pltpu.make_async_copy(
            ot,
            o_hbm.at[pl.ds(NFULL * (TCH // 4), TAIL // 4)],
            tsem,
        ).start()
        for c in (NFULL - 2, NFULL - 1):
            kb = c % 2
            pltpu.make_async_copy(
                ov.at[kb],
                o_hbm.at[pl.ds(c * (TCH // 4), TCH // 4)],
                osems.at[kb],
            ).wait()
        pltpu.make_async_copy(
            ot,
            o_hbm.at[pl.ds(NFULL * (TCH // 4), TAIL // 4)],
            tsem,
        ).wait()

    return pl.pallas_call(
        body,
        in_specs=[pl.BlockSpec(memory_space=pltpu.HBM)],
        out_specs=pl.BlockSpec(memory_space=pltpu.HBM),
        out_shape=jax.ShapeDtypeStruct(
            (NUM_EMB * DIM // (4 * DIM), 4 * DIM), jnp.float32
        ),
        scratch_shapes=[
            pltpu.VMEM((2, DIM, TCH), jnp.float32),
            pltpu.VMEM((2, TCH // 4, 4 * DIM), jnp.float32),
            pltpu.VMEM((DIM, TAIL), jnp.float32),
            pltpu.VMEM((TAIL // 4, 4 * DIM), jnp.float32),
            pltpu.SemaphoreType.DMA((2,)),
            pltpu.SemaphoreType.DMA((2,)),
            pltpu.SemaphoreType.DMA,
        ],
    )(tt2)


def _sc_assemble(table, idx, dst_sp, dense_rows, dst_de):
    mesh = plsc.VectorSubcoreMesh(core_axis_name="c", subcore_axis_name="s")

    @functools.partial(
        pl.kernel,
        out_type=jax.ShapeDtypeStruct((B * NROW, DIM), jnp.float32),
        mesh=mesh,
        scratch_types=[
            pltpu.VMEM((NG_SP, G), jnp.int32),
            pltpu.VMEM((NG_SP, G), jnp.int32),
            pltpu.VMEM((NG_DE, G), jnp.int32),
            pltpu.VMEM((NG_SP * G, DIM), jnp.float32),
            pltpu.SemaphoreType.DMA,
            pltpu.SemaphoreType.DMA,
        ],
        compiler_params=pltpu.CompilerParams(use_tc_tiling_on_sc=False),
    )
    def k(table_hbm, idx_hbm, dst_sp_hbm, dense_hbm, dst_de_hbm, out_hbm,
          idx_v, dsp_v, dde_v, rows_v, gsem, ssem):
        wid = lax.axis_index("s") * NC + lax.axis_index("c")
        pltpu.sync_copy(idx_hbm.at[wid], idx_v)
        pltpu.sync_copy(dst_sp_hbm.at[wid], dsp_v)
        pltpu.sync_copy(dst_de_hbm.at[wid], dde_v)

        def fire_gather(g, _):
            pltpu.async_copy(
                table_hbm.at[idx_v.at[g]],
                rows_v.at[pl.ds(g * G, G)],
                gsem,
            )
            return _

        lax.fori_loop(0, NG_SP, fire_gather, None)

        def drain_gather(g, _):
            pltpu.make_async_copy(
                table_hbm.at[idx_v.at[g]],
                rows_v.at[pl.ds(g * G, G)],
                gsem,
            ).wait()
            return _

        lax.fori_loop(0, NG_SP, drain_gather, None)

        def fire_scatter(g, _):
            pltpu.async_copy(
                rows_v.at[pl.ds(g * G, G)],
                out_hbm.at[dsp_v.at[g]],
                ssem,
            )
            return _

        lax.fori_loop(0, NG_SP, fire_scatter, None)

        def drain_scatter(g, _):
            pltpu.make_async_copy(
                rows_v.at[pl.ds(g * G, G)],
                out_hbm.at[dsp_v.at[g]],
                ssem,
            ).wait()
            return _

        lax.fori_loop(0, NG_SP, drain_scatter, None)

        nde = NG_DE * G
        pltpu.sync_copy(
            dense_hbm.at[pl.ds(wid * nde, nde)], rows_v.at[pl.ds(0, nde)]
        )

        def fire_dense(g, _):
            pltpu.async_copy(
                rows_v.at[pl.ds(g * G, G)],
                out_hbm.at[dde_v.at[g]],
                ssem,
            )
            return _

        lax.fori_loop(0, NG_DE, fire_dense, None)

        def drain_dense(g, _):
            pltpu.make_async_copy(
                rows_v.at[pl.ds(g * G, G)],
                out_hbm.at[dde_v.at[g]],
                ssem,
            ).wait()
            return _

        lax.fori_loop(0, NG_DE, drain_dense, None)

    return k(table, idx, dst_sp, dense_rows, dst_de)


def kernel(sparse_inputs, dense_inputs, emb_table, W, b):
    table = _transpose_tc(emb_table.T).reshape(NUM_EMB, DIM)
    dense_rows = _dense_tc(dense_inputs, W, b).reshape(B * ND, DIM)

    r = sparse_inputs.astype(jnp.int32)
    c = r // TCH
    l = r - c * TCH
    q = jnp.where(c < NUM_EMB // TCH, TCH // 4, (NUM_EMB % TCH) // 4)
    ridx = c * TCH + 4 * (l % q) + l // q
    idx = ridx.reshape(NW, NG_SP, G)

    i = jnp.arange(B * NSF, dtype=jnp.int32)
    dst_sp = (i + ND * (i // NSF)).reshape(NW, NG_SP, G)
    j = jnp.arange(B * ND, dtype=jnp.int32)
    dst_de = (NSF + (j // ND) * NROW + j % ND).reshape(NW, NG_DE, G)

    out_flat = _sc_assemble(table, idx, dst_sp, dense_rows, dst_de)
    return lax.optimization_barrier(out_flat.reshape(B, NROW, DIM))


# speedup vs baseline: 1.6069x; 1.6069x over previous
import functools

import jax
import jax.numpy as jnp
from jax import lax
from jax.experimental import pallas as pl
from jax.experimental.pallas import tpu as pltpu
from jax.experimental.pallas import tpu_sc as plsc

NUM_EMB = 1000000
DIM = 32
ND = 13
NSF = 26
B = 4096
NROW = NSF + ND

NC = 2
NS = 16
NW = NC * NS
BPW = B // NW
G = 128
NG_SP = BPW * NSF // G
NG_DE = BPW * ND // G

TCH = 32000


def _dense_tc(x, w, bias):
    def body(x_ref, w_ref, b_ref, o_ref):
        o_ref[...] = (
            lax.dot_general(
                x_ref[...], w_ref[...],
                dimension_numbers=(((1,), (1,)), ((), ())),
                preferred_element_type=jnp.float32,
            )
            + b_ref[...]
        )

    return pl.pallas_call(
        body,
        out_shape=jax.ShapeDtypeStruct((B, ND * DIM), jnp.float32),
    )(x, w, bias.reshape(1, ND * DIM))


def _transpose_tc(tt2):
    NFULL = NUM_EMB // TCH
    TAIL = NUM_EMB - TCH * NFULL

    def body(x_hbm, o_hbm, xv, ov, xt, ot, isems, osems, tsem):
        def fire_in(c):
            pltpu.make_async_copy(
                x_hbm.at[:, pl.ds(c * TCH, TCH)],
                xv.at[c % 2],
                isems.at[c % 2],
            ).start()

        fire_in(0)
        fire_in(1)
        pltpu.make_async_copy(
            x_hbm.at[:, pl.ds(NFULL * TCH, TAIL)], xt, tsem
        ).start()
        for c in range(NFULL):
            kb = c % 2
            pltpu.make_async_copy(
                x_hbm.at[:, pl.ds(c * TCH, TCH)], xv.at[kb], isems.at[kb]
            ).wait()
            q = TCH // 4
            y = jnp.concatenate(
                [xv[kb][:, j * q:(j + 1) * q] for j in range(4)], axis=0
            ).T
            if c >= 2:
                pltpu.make_async_copy(
                    ov.at[kb],
                    o_hbm.at[pl.ds((c - 2) * (TCH // 4), TCH // 4)],
                    osems.at[kb],
                ).wait()
            ov[kb] = y
            pltpu.make_async_copy(
                ov.at[kb],
                o_hbm.at[pl.ds(c * (TCH // 4), TCH // 4)],
                osems.at[kb],
            ).start()
            if c + 2 < NFULL:
                fire_in(c + 2)
        pltpu.make_async_copy(
            x_hbm.at[:, pl.ds(NFULL * TCH, TAIL)], xt, tsem
        ).wait()
        qt = TAIL // 4
        ot[...] = jnp.concatenate(
            [xt[:, j * qt:(j + 1) * qt] for j in range(4)], axis=0
        ).T
        pltpu.make_async_copy(
            ot,
            o_hbm.at[pl.ds(NFULL * (TCH // 4), TAIL // 4)],
            tsem,
        ).start()
        for c in (NFULL - 2, NFULL - 1):
            kb = c % 2
            pltpu.make_async_copy(
                ov.at[kb],
                o_hbm.at[pl.ds(c * (TCH // 4), TCH // 4)],
                osems.at[kb],
            ).wait()
        pltpu.make_async_copy(
            ot,
            o_hbm.at[pl.ds(NFULL * (TCH // 4), TAIL // 4)],
            tsem,
        ).wait()

    return pl.pallas_call(
        body,
        in_specs=[pl.BlockSpec(memory_space=pltpu.HBM)],
        out_specs=pl.BlockSpec(memory_space=pltpu.HBM),
        out_shape=jax.ShapeDtypeStruct(
            (NUM_EMB * DIM // (4 * DIM), 4 * DIM), jnp.float32
        ),
        scratch_shapes=[
            pltpu.VMEM((2, DIM, TCH), jnp.float32),
            pltpu.VMEM((2, TCH // 4, 4 * DIM), jnp.float32),
            pltpu.VMEM((DIM, TAIL), jnp.float32),
            pltpu.VMEM((TAIL // 4, 4 * DIM), jnp.float32),
            pltpu.SemaphoreType.DMA((2,)),
            pltpu.SemaphoreType.DMA((2,)),
            pltpu.SemaphoreType.DMA,
        ],
    )(tt2)


def _sc_assemble(table, idx, dst_sp, dense_rows, dst_de):
    mesh = plsc.VectorSubcoreMesh(core_axis_name="c", subcore_axis_name="s")

    @functools.partial(
        pl.kernel,
        out_type=jax.ShapeDtypeStruct((B * NROW, DIM), jnp.float32),
        mesh=mesh,
        scratch_types=[
            pltpu.VMEM((NG_SP, G), jnp.int32),
            pltpu.VMEM((NG_SP, G), jnp.int32),
            pltpu.VMEM((NG_DE, G), jnp.int32),
            pltpu.VMEM((NG_SP * G, DIM), jnp.float32),
            pltpu.SemaphoreType.DMA,
            pltpu.SemaphoreType.DMA,
        ],
        compiler_params=pltpu.CompilerParams(use_tc_tiling_on_sc=False),
    )
    def k(table_hbm, idx_hbm, dst_sp_hbm, dense_hbm, dst_de_hbm, out_hbm,
          idx_v, dsp_v, dde_v, rows_v, gsem, ssem):
        wid = lax.axis_index("s") * NC + lax.axis_index("c")
        pltpu.sync_copy(idx_hbm.at[wid], idx_v)
        pltpu.sync_copy(dst_sp_hbm.at[wid], dsp_v)
        pltpu.sync_copy(dst_de_hbm.at[wid], dde_v)

        def fire_gather(g, _):
            pltpu.async_copy(
                table_hbm.at[idx_v.at[g]],
                rows_v.at[pl.ds(g * G, G)],
                gsem,
            )
            return _

        lax.fori_loop(0, NG_SP, fire_gather, None)

        def drain_gather(g, _):
            pltpu.make_async_copy(
                table_hbm.at[idx_v.at[g]],
                rows_v.at[pl.ds(g * G, G)],
                gsem,
            ).wait()
            return _

        lax.fori_loop(0, NG_SP, drain_gather, None)

        def fire_scatter(g, _):
            pltpu.async_copy(
                rows_v.at[pl.ds(g * G, G)],
                out_hbm.at[dsp_v.at[g]],
                ssem,
            )
            return _

        lax.fori_loop(0, NG_SP, fire_scatter, None)

        def drain_scatter(g, _):
            pltpu.make_async_copy(
                rows_v.at[pl.ds(g * G, G)],
                out_hbm.at[dsp_v.at[g]],
                ssem,
            ).wait()
            return _

        lax.fori_loop(0, NG_SP, drain_scatter, None)

        nde = NG_DE * G
        pltpu.sync_copy(
            dense_hbm.at[pl.ds(wid * nde, nde)], rows_v.at[pl.ds(0, nde)]
        )

        def fire_dense(g, _):
            pltpu.async_copy(
                rows_v.at[pl.ds(g * G, G)],
                out_hbm.at[dde_v.at[g]],
                ssem,
            )
            return _

        lax.fori_loop(0, NG_DE, fire_dense, None)

        def drain_dense(g, _):
            pltpu.make_async_copy(
                rows_v.at[pl.ds(g * G, G)],
                out_hbm.at[dde_v.at[g]],
                ssem,
            ).wait()
            return _

        lax.fori_loop(0, NG_DE, drain_dense, None)

    return k(table, idx, dst_sp, dense_rows, dst_de)


def kernel(sparse_inputs, dense_inputs, emb_table, W, b):
    table = _transpose_tc(emb_table.T).reshape(NUM_EMB, DIM)
    dense_rows = _dense_tc(dense_inputs, W, b).reshape(B * ND, DIM)

    r = sparse_inputs.astype(jnp.int32)
    c = r // TCH
    l = r - c * TCH
    q = jnp.where(c < NUM_EMB // TCH, TCH // 4, (NUM_EMB % TCH) // 4)
    ridx = c * TCH + 4 * (l % q) + l // q
    idx = ridx.reshape(NW, NG_SP, G)

    i = jnp.arange(B * NSF, dtype=jnp.int32)
    dst_sp = (i + ND * (i // NSF)).reshape(NW, NG_SP, G)
    j = jnp.arange(B * ND, dtype=jnp.int32)
    dst_de = (NSF + (j // ND) * NROW + j % ND).reshape(NW, NG_DE, G)

    out_flat = _sc_assemble(table, idx, dst_sp, dense_rows, dst_de)
    return out_flat.reshape(B, NROW, DIM)
